# baseline (speedup 1.0000x reference)
import functools

import jax
import jax.numpy as jnp
from jax import lax
from jax.experimental import pallas as pl
from jax.experimental.pallas import tpu as pltpu
from jax.experimental.pallas import tpu_sc as plsc

B = 64
H = 512
W = 256
C = 3
V = 14475
P = H * W
BSTRIDE = P * C

NC = 2
NS = 16
L = 16
NW = NC * NS

NV = 464
VPAD = NW * NV
FL = NV * C
NCH = FL // L


@functools.partial(
    pl.kernel,
    out_type=jax.ShapeDtypeStruct((B, VPAD * C), jnp.float32),
    mesh=plsc.VectorSubcoreMesh(core_axis_name="c", subcore_axis_name="s",
                                num_cores=NC, num_subcores=NS),
    compiler_params=pltpu.CompilerParams(use_tc_tiling_on_sc=False,
                                         needs_layout_passes=False),
    scratch_types=[
        pltpu.VMEM((4 * NV,), jnp.int32),
        pltpu.VMEM((NV,), jnp.int32),
        pltpu.VMEM((NV,), jnp.int32),
        pltpu.VMEM((FL,), jnp.int32),
        pltpu.VMEM((FL,), jnp.int32),
        pltpu.VMEM((FL,), jnp.int32),
        pltpu.VMEM((FL,), jnp.int32),
        pltpu.VMEM((FL,), jnp.float32),
        pltpu.VMEM((FL,), jnp.float32),
        pltpu.VMEM((FL,), jnp.float32),
        pltpu.SemaphoreType.DMA,
        pltpu.SemaphoreType.DMA,
    ],
)
def _uv2mesh_sc(uv_hbm, hw_hbm, out_hbm,
                hw_v, idx0, idx1, idxf0, idxf1, idx0b, idx1b, g0, g1, ob,
                sem0, sem1):
    wid = lax.axis_index("s") * NC + lax.axis_index("c")
    v0 = wid * NV

    for j in range(4):
        pltpu.sync_copy(hw_hbm.at[pl.ds(j * VPAD + v0, NV)],
                        hw_v.at[pl.ds(j * NV, NV)])

    iota = lax.iota(jnp.int32, L)
    for i in range(NV // L):
        s = pl.ds(i * L, L)
        idx0[s] = hw_v[pl.ds(0 * NV + i * L, L)] * W + hw_v[pl.ds(1 * NV + i * L, L)]
        idx1[s] = hw_v[pl.ds(2 * NV + i * L, L)] * W + hw_v[pl.ds(3 * NV + i * L, L)]

    for k in range(NCH):
        s = pl.ds(k * L, L)
        p = iota + (k * L)
        r = lax.shift_right_logical(p * 21846, 16)
        cc = p - r * C
        idxf0[s] = plsc.load_gather(idx0, [r]) * C + cc
        idxf1[s] = plsc.load_gather(idx1, [r]) * C + cc

    def body(b, carry):
        boff = b * BSTRIDE
        for k in range(NCH):
            s = pl.ds(k * L, L)
            idx0b[s] = idxf0[s] + boff
            idx1b[s] = idxf1[s] + boff
        cp0 = pltpu.async_copy(uv_hbm.at[idx0b], g0, sem0)
        cp1 = pltpu.async_copy(uv_hbm.at[idx1b], g1, sem1)
        cp0.wait()
        cp1.wait()
        for k in range(NCH):
            s = pl.ds(k * L, L)
            ob[s] = (g0[s] + g1[s]) * 0.5
        pltpu.sync_copy(ob, out_hbm.at[b, pl.ds(v0 * C, FL)])
        return carry

    lax.fori_loop(0, B, body, None)


def kernel(uv, uv_pixels):
    uv_flat = uv.reshape(B * H * W * C)
    hp = uv_pixels.astype(jnp.int32)
    hw = jnp.stack([hp[:, 0, 0], hp[:, 0, 1], hp[:, 1, 0], hp[:, 1, 1]])
    hw = jnp.pad(hw, ((0, 0), (0, VPAD - V))).reshape(4 * VPAD)
    out = _uv2mesh_sc(uv_flat, hw)
    return out.reshape(B, VPAD, C)[:, :V, :]


# speedup vs baseline: 15.7154x; 15.7154x over previous
import functools

import jax
import jax.numpy as jnp
from jax import lax
from jax.experimental import pallas as pl
from jax.experimental.pallas import tpu as pltpu
from jax.experimental.pallas import tpu_sc as plsc

B = 64
H = 512
W = 256
C = 3
V = 14475
P = H * W
BSTRIDE = P * C

NC = 2
NS = 16
L = 16
NW = NC * NS

NV = 464
VPAD = NW * NV
FL = NV * C
NCH = FL // L


@functools.partial(
    pl.kernel,
    out_type=jax.ShapeDtypeStruct((B, VPAD * C), jnp.float32),
    mesh=plsc.VectorSubcoreMesh(core_axis_name="c", subcore_axis_name="s",
                                num_cores=NC, num_subcores=NS),
    compiler_params=pltpu.CompilerParams(use_tc_tiling_on_sc=False,
                                         needs_layout_passes=False),
    scratch_types=[
        pltpu.VMEM((4 * NV,), jnp.int32),
        pltpu.VMEM((NV,), jnp.int32),
        pltpu.VMEM((NV,), jnp.int32),
        pltpu.VMEM((FL,), jnp.int32),
        pltpu.VMEM((FL,), jnp.int32),
        pltpu.VMEM((FL,), jnp.int32),
        pltpu.VMEM((FL,), jnp.int32),
        pltpu.VMEM((FL,), jnp.float32),
        pltpu.VMEM((FL,), jnp.float32),
        pltpu.VMEM((FL,), jnp.float32),
        pltpu.SemaphoreType.DMA,
        pltpu.SemaphoreType.DMA,
    ],
)
def _uv2mesh_sc(uv_hbm, hw_hbm, out_hbm,
                hw_v, idx0, idx1, idxf0, idxf1, idx0b, idx1b, g0, g1, ob,
                sem0, sem1):
    wid = lax.axis_index("s") * NC + lax.axis_index("c")
    v0 = wid * NV

    for j in range(4):
        pltpu.sync_copy(hw_hbm.at[pl.ds(j * VPAD + v0, NV)],
                        hw_v.at[pl.ds(j * NV, NV)])

    iota = lax.iota(jnp.int32, L)
    for i in range(NV // L):
        s = pl.ds(i * L, L)
        h0 = hw_v[pl.ds(0 * NV + i * L, L)]
        w0 = hw_v[pl.ds(1 * NV + i * L, L)]
        h1 = hw_v[pl.ds(2 * NV + i * L, L)]
        w1 = hw_v[pl.ds(3 * NV + i * L, L)]
        t0 = (lax.shift_left(lax.shift_right_logical(h0, 3), 11)
              + lax.shift_left(h0 & 1, 10)
              + lax.shift_left(lax.shift_right_logical(w0, 7), 9)
              + lax.shift_left(lax.shift_right_logical(h0, 1) & 3, 7)
              + (w0 & 127))
        t1 = (lax.shift_left(lax.shift_right_logical(h1, 3), 11)
              + lax.shift_left(h1 & 1, 10)
              + lax.shift_left(lax.shift_right_logical(w1, 7), 9)
              + lax.shift_left(lax.shift_right_logical(h1, 1) & 3, 7)
              + (w1 & 127))
        idx0[s] = t0
        idx1[s] = t1

    for k in range(NCH):
        s = pl.ds(k * L, L)
        p = iota + (k * L)
        r = lax.shift_right_logical(p * 21846, 16)
        cc = p - r * C
        coff = lax.shift_left(cc, 17)
        idxf0[s] = plsc.load_gather(idx0, [r]) + coff
        idxf1[s] = plsc.load_gather(idx1, [r]) + coff

    def body(b, carry):
        boff = b * BSTRIDE
        for k in range(NCH):
            s = pl.ds(k * L, L)
            idx0b[s] = idxf0[s] + boff
            idx1b[s] = idxf1[s] + boff
        cp0 = pltpu.async_copy(uv_hbm.at[idx0b], g0, sem0)
        cp1 = pltpu.async_copy(uv_hbm.at[idx1b], g1, sem1)
        cp0.wait()
        cp1.wait()
        for k in range(NCH):
            s = pl.ds(k * L, L)
            ob[s] = (g0[s] + g1[s]) * 0.5
        pltpu.sync_copy(ob, out_hbm.at[b, pl.ds(v0 * C, FL)])
        return carry

    lax.fori_loop(0, B, body, None)


def kernel(uv, uv_pixels):
    uv_flat = (uv.transpose(0, 3, 1, 2)
                 .reshape(B, C, H // 8, 4, 2, 2, 128)
                 .transpose(0, 1, 2, 4, 5, 3, 6)
                 .reshape(B * C * H * W))
    hp = uv_pixels.astype(jnp.int32)
    hw = jnp.stack([hp[:, 0, 0], hp[:, 0, 1], hp[:, 1, 0], hp[:, 1, 1]])
    hw = jnp.pad(hw, ((0, 0), (0, VPAD - V))).reshape(4 * VPAD)
    out = _uv2mesh_sc(uv_flat, hw)
    return out.reshape(B, VPAD, C)[:, :V, :]


# speedup vs baseline: 18.2109x; 1.1588x over previous
import functools

import jax
import jax.numpy as jnp
from jax import lax
from jax.experimental import pallas as pl
from jax.experimental.pallas import tpu as pltpu
from jax.experimental.pallas import tpu_sc as plsc

B = 64
H = 512
W = 256
C = 3
V = 14475
P = H * W
BSTRIDE = P * C

NC = 2
NS = 16
L = 16
NW = NC * NS

NV = 464
VPAD = NW * NV
FL = NV * C
NCH = FL // L


@functools.partial(
    pl.kernel,
    out_type=jax.ShapeDtypeStruct((B, VPAD * C), jnp.float32),
    mesh=plsc.VectorSubcoreMesh(core_axis_name="c", subcore_axis_name="s",
                                num_cores=NC, num_subcores=NS),
    compiler_params=pltpu.CompilerParams(use_tc_tiling_on_sc=False,
                                         needs_layout_passes=False),
    scratch_types=[
        pltpu.VMEM((4 * NV,), jnp.int32),
        pltpu.VMEM((NV,), jnp.int32),
        pltpu.VMEM((NV,), jnp.int32),
        pltpu.VMEM((FL,), jnp.int32),
        pltpu.VMEM((FL,), jnp.int32),
        pltpu.VMEM((FL,), jnp.float32),
        pltpu.VMEM((FL,), jnp.float32),
        pltpu.VMEM((FL,), jnp.float32),
        pltpu.VMEM((FL,), jnp.float32),
        pltpu.VMEM((FL,), jnp.float32),
        pltpu.VMEM((FL,), jnp.float32),
        pltpu.SemaphoreType.DMA,
        pltpu.SemaphoreType.DMA,
        pltpu.SemaphoreType.DMA,
        pltpu.SemaphoreType.DMA,
    ],
)
def _uv2mesh_sc(uv_hbm, hw_hbm, out_hbm,
                hw_v, idx0, idx1, idxf0, idxf1,
                g00, g10, g01, g11, ob0, ob1,
                sga, sgb, soa, sob):
    wid = lax.axis_index("s") * NC + lax.axis_index("c")
    v0 = wid * NV

    for j in range(4):
        pltpu.sync_copy(hw_hbm.at[pl.ds(j * VPAD + v0, NV)],
                        hw_v.at[pl.ds(j * NV, NV)])

    iota = lax.iota(jnp.int32, L)
    for i in range(NV // L):
        s = pl.ds(i * L, L)
        h0 = hw_v[pl.ds(0 * NV + i * L, L)]
        w0 = hw_v[pl.ds(1 * NV + i * L, L)]
        h1 = hw_v[pl.ds(2 * NV + i * L, L)]
        w1 = hw_v[pl.ds(3 * NV + i * L, L)]
        t0 = (lax.shift_left(lax.shift_right_logical(h0, 3), 11)
              + lax.shift_left(h0 & 1, 10)
              + lax.shift_left(lax.shift_right_logical(w0, 7), 9)
              + lax.shift_left(lax.shift_right_logical(h0, 1) & 3, 7)
              + (w0 & 127))
        t1 = (lax.shift_left(lax.shift_right_logical(h1, 3), 11)
              + lax.shift_left(h1 & 1, 10)
              + lax.shift_left(lax.shift_right_logical(w1, 7), 9)
              + lax.shift_left(lax.shift_right_logical(h1, 1) & 3, 7)
              + (w1 & 127))
        idx0[s] = t0
        idx1[s] = t1

    for k in range(NCH):
        s = pl.ds(k * L, L)
        p = iota + (k * L)
        r = lax.shift_right_logical(p * 21846, 16)
        cc = p - r * C
        coff = lax.shift_left(cc, 17)
        idxf0[s] = plsc.load_gather(idx0, [r]) + coff
        idxf1[s] = plsc.load_gather(idx1, [r]) + coff

    def issue(b, ga, gb, sem):
        src = uv_hbm.at[pl.ds(b * BSTRIDE, BSTRIDE)]
        pltpu.async_copy(src.at[idxf0], ga, sem)
        pltpu.async_copy(src.at[idxf1], gb, sem)

    def drain(ga, gb, sem):
        pltpu.make_async_copy(uv_hbm.at[idxf0], ga, sem).wait()
        pltpu.make_async_copy(uv_hbm.at[idxf1], gb, sem).wait()

    def compute_and_store(b, ga, gb, ob, so, first):
        @pl.when(jnp.logical_not(first))
        def _():
            pltpu.make_async_copy(ob, out_hbm.at[0, pl.ds(v0 * C, FL)],
                                  so).wait()
        for k in range(NCH):
            s = pl.ds(k * L, L)
            ob[s] = (ga[s] + gb[s]) * 0.5
        pltpu.async_copy(ob, out_hbm.at[b, pl.ds(v0 * C, FL)], so)

    issue(0, g00, g10, sga)

    def body(i, carry):
        b0 = i * 2
        b1 = b0 + 1
        issue(b1, g01, g11, sgb)
        drain(g00, g10, sga)
        compute_and_store(b0, g00, g10, ob0, soa, i == 0)
        @pl.when(b0 + 2 < B)
        def _():
            issue(b0 + 2, g00, g10, sga)
        drain(g01, g11, sgb)
        compute_and_store(b1, g01, g11, ob1, sob, i == 0)
        return carry

    lax.fori_loop(0, B // 2, body, None)
    pltpu.make_async_copy(ob0, out_hbm.at[0, pl.ds(v0 * C, FL)], soa).wait()
    pltpu.make_async_copy(ob1, out_hbm.at[0, pl.ds(v0 * C, FL)], sob).wait()


def kernel(uv, uv_pixels):
    uv_flat = (uv.transpose(0, 3, 1, 2)
                 .reshape(B, C, H // 8, 4, 2, 2, 128)
                 .transpose(0, 1, 2, 4, 5, 3, 6)
                 .reshape(B * C * H * W))
    hp = uv_pixels.astype(jnp.int32)
    hw = jnp.stack([hp[:, 0, 0], hp[:, 0, 1], hp[:, 1, 0], hp[:, 1, 1]])
    hw = jnp.pad(hw, ((0, 0), (0, VPAD - V))).reshape(4 * VPAD)
    out = _uv2mesh_sc(uv_flat, hw)
    return out.reshape(B, VPAD, C)[:, :V, :]


# speedup vs baseline: 19.3747x; 1.0639x over previous
import functools

import jax
import jax.numpy as jnp
from jax import lax
from jax.experimental import pallas as pl
from jax.experimental.pallas import tpu as pltpu
from jax.experimental.pallas import tpu_sc as plsc

B = 64
H = 512
W = 256
C = 3
V = 14475
P = H * W
BSTRIDE = P * C

NC = 2
NS = 16
L = 16
NW = NC * NS

NV = 464
VPAD = NW * NV
FL = NV * C
NCH = FL // L
NSLOT = 4


@functools.partial(
    pl.kernel,
    out_type=jax.ShapeDtypeStruct((B, VPAD * C), jnp.float32),
    mesh=plsc.VectorSubcoreMesh(core_axis_name="c", subcore_axis_name="s",
                                num_cores=NC, num_subcores=NS),
    compiler_params=pltpu.CompilerParams(use_tc_tiling_on_sc=False,
                                         needs_layout_passes=False),
    scratch_types=(
        [pltpu.VMEM((4 * NV,), jnp.int32)]
        + [pltpu.VMEM((NV,), jnp.int32)] * 2
        + [pltpu.VMEM((2 * FL,), jnp.int32)]
        + [pltpu.VMEM((2 * FL,), jnp.float32)] * NSLOT
        + [pltpu.VMEM((FL,), jnp.float32)] * NSLOT
        + [pltpu.SemaphoreType.DMA] * NSLOT
        + [pltpu.SemaphoreType.DMA] * NSLOT
    ),
)
def _uv2mesh_sc(uv_hbm, hw_hbm, out_hbm, hw_v, idx0, idx1, idxf, *rest):
    g = rest[0:NSLOT]
    ob = rest[NSLOT:2 * NSLOT]
    sg = rest[2 * NSLOT:3 * NSLOT]
    so = rest[3 * NSLOT:4 * NSLOT]

    wid = lax.axis_index("s") * NC + lax.axis_index("c")
    v0 = wid * NV

    for j in range(4):
        pltpu.sync_copy(hw_hbm.at[pl.ds(j * VPAD + v0, NV)],
                        hw_v.at[pl.ds(j * NV, NV)])

    iota = lax.iota(jnp.int32, L)
    for i in range(NV // L):
        s = pl.ds(i * L, L)
        h0 = hw_v[pl.ds(0 * NV + i * L, L)]
        w0 = hw_v[pl.ds(1 * NV + i * L, L)]
        h1 = hw_v[pl.ds(2 * NV + i * L, L)]
        w1 = hw_v[pl.ds(3 * NV + i * L, L)]
        t0 = (lax.shift_left(lax.shift_right_logical(h0, 3), 11)
              + lax.shift_left(h0 & 1, 10)
              + lax.shift_left(lax.shift_right_logical(w0, 7), 9)
              + lax.shift_left(lax.shift_right_logical(h0, 1) & 3, 7)
              + (w0 & 127))
        t1 = (lax.shift_left(lax.shift_right_logical(h1, 3), 11)
              + lax.shift_left(h1 & 1, 10)
              + lax.shift_left(lax.shift_right_logical(w1, 7), 9)
              + lax.shift_left(lax.shift_right_logical(h1, 1) & 3, 7)
              + (w1 & 127))
        idx0[s] = t0
        idx1[s] = t1

    for k in range(NCH):
        s = pl.ds(k * L, L)
        s1 = pl.ds(FL + k * L, L)
        p = iota + (k * L)
        r = lax.shift_right_logical(p * 21846, 16)
        cc = p - r * C
        coff = lax.shift_left(cc, 17)
        idxf[s] = plsc.load_gather(idx0, [r]) + coff
        idxf[s1] = plsc.load_gather(idx1, [r]) + coff

    def issue(b, slot):
        src = uv_hbm.at[pl.ds(b * BSTRIDE, BSTRIDE)]
        pltpu.async_copy(src.at[idxf], g[slot], sg[slot])

    def step(b, slot, first):
        pltpu.make_async_copy(uv_hbm.at[idxf], g[slot], sg[slot]).wait()
        @pl.when(jnp.logical_not(first))
        def _():
            pltpu.make_async_copy(ob[slot], out_hbm.at[0, pl.ds(v0 * C, FL)],
                                  so[slot]).wait()
        ga = g[slot]
        obs = ob[slot]
        for k in range(NCH):
            s = pl.ds(k * L, L)
            s1 = pl.ds(FL + k * L, L)
            obs[s] = (ga[s] + ga[s1]) * 0.5
        pltpu.async_copy(obs, out_hbm.at[b, pl.ds(v0 * C, FL)], so[slot])
        @pl.when(b + NSLOT < B)
        def _():
            issue(b + NSLOT, slot)

    for j in range(NSLOT):
        issue(j, j)

    def body(i, carry):
        b0 = i * NSLOT
        for j in range(NSLOT):
            step(b0 + j, j, i == 0)
        return carry

    lax.fori_loop(0, B // NSLOT, body, None)
    for j in range(NSLOT):
        pltpu.make_async_copy(ob[j], out_hbm.at[0, pl.ds(v0 * C, FL)],
                              so[j]).wait()


def kernel(uv, uv_pixels):
    uv_flat = (uv.transpose(0, 3, 1, 2)
                 .reshape(B, C, H // 8, 4, 2, 2, 128)
                 .transpose(0, 1, 2, 4, 5, 3, 6)
                 .reshape(B * C * H * W))
    hp = uv_pixels.astype(jnp.int32)
    hw = jnp.stack([hp[:, 0, 0], hp[:, 0, 1], hp[:, 1, 0], hp[:, 1, 1]])
    hw = jnp.pad(hw, ((0, 0), (0, VPAD - V))).reshape(4 * VPAD)
    out = _uv2mesh_sc(uv_flat, hw)
    return out.reshape(B, VPAD, C)[:, :V, :]


# speedup vs baseline: 24.8386x; 1.2820x over previous
import functools

import jax
import jax.numpy as jnp
from jax import lax
from jax.experimental import pallas as pl
from jax.experimental.pallas import tpu as pltpu
from jax.experimental.pallas import tpu_sc as plsc

B = 64
H = 512
W = 256
C = 3
V = 14475
P = H * W
BSTRIDE = P * C

NC = 2
NS = 16
L = 16
NW = NC * NS

VT = 114
VPAD = VT * 128
NV = 512
FL = NV * C
NCH = FL // L
NSLOT = 4


@functools.partial(
    pl.kernel,
    out_type=jax.ShapeDtypeStruct((C * 8, VT, 8 * 128), jnp.float32),
    mesh=plsc.VectorSubcoreMesh(core_axis_name="c", subcore_axis_name="s",
                                num_cores=NC, num_subcores=NS),
    compiler_params=pltpu.CompilerParams(use_tc_tiling_on_sc=False,
                                         needs_layout_passes=False),
    scratch_types=(
        [pltpu.VMEM((4 * NV,), jnp.int32)]
        + [pltpu.VMEM((NV,), jnp.int32)] * 2
        + [pltpu.VMEM((2 * FL,), jnp.int32)]
        + [pltpu.VMEM((2 * FL,), jnp.float32)] * NSLOT
        + [pltpu.VMEM((C * 4, 128), jnp.float32)] * NSLOT
        + [pltpu.SemaphoreType.DMA] * NSLOT
        + [pltpu.SemaphoreType.DMA] * NSLOT
    ),
)
def _uv2mesh_sc(uv_hbm, hw_hbm, out_hbm, hw_v, idx0, idx1, idxf, *rest):
    g = rest[0:NSLOT]
    ob = rest[NSLOT:2 * NSLOT]
    sg = rest[2 * NSLOT:3 * NSLOT]
    so = rest[3 * NSLOT:4 * NSLOT]

    wid = lax.axis_index("s") * NC + lax.axis_index("c")
    vt0 = lax.shift_right_logical(wid * VT, 5)
    vt1 = lax.shift_right_logical((wid + 1) * VT, 5)
    has4 = (vt1 - vt0) == 4
    v0 = pl.multiple_of(lax.shift_left(vt0, 7), 128)

    for j in range(4):
        pltpu.sync_copy(hw_hbm.at[pl.ds(j * VPAD + v0, NV)],
                        hw_v.at[pl.ds(j * NV, NV)])

    for i in range(NV // L):
        s = pl.ds(i * L, L)
        h0 = hw_v[pl.ds(0 * NV + i * L, L)]
        w0 = hw_v[pl.ds(1 * NV + i * L, L)]
        h1 = hw_v[pl.ds(2 * NV + i * L, L)]
        w1 = hw_v[pl.ds(3 * NV + i * L, L)]
        t0 = (lax.shift_left(lax.shift_right_logical(h0, 3), 11)
              + lax.shift_left(h0 & 1, 10)
              + lax.shift_left(lax.shift_right_logical(w0, 7), 9)
              + lax.shift_left(lax.shift_right_logical(h0, 1) & 3, 7)
              + (w0 & 127))
        t1 = (lax.shift_left(lax.shift_right_logical(h1, 3), 11)
              + lax.shift_left(h1 & 1, 10)
              + lax.shift_left(lax.shift_right_logical(w1, 7), 9)
              + lax.shift_left(lax.shift_right_logical(h1, 1) & 3, 7)
              + (w1 & 127))
        idx0[s] = t0
        idx1[s] = t1

    for c in range(C):
        for i in range(NV // L):
            s = pl.ds(c * NV + i * L, L)
            s1 = pl.ds(FL + c * NV + i * L, L)
            vs = pl.ds(i * L, L)
            idxf[s] = idx0[vs] + (c * P)
            idxf[s1] = idx1[vs] + (c * P)

    def issue(b, slot):
        src = uv_hbm.at[pl.ds(b * BSTRIDE, BSTRIDE)]
        pltpu.async_copy(src.at[idxf], g[slot], sg[slot])

    def out_writes(b, slot, do_issue):
        bh = lax.shift_right_logical(b, 3)
        bl = (b & 7) * 128
        obs = ob[slot]
        sem = so[slot]
        for c in range(C):
            dst3 = out_hbm.at[c * 8 + bh, pl.ds(vt0, 3), pl.ds(bl, 128)]
            src3 = obs.at[pl.ds(c * 4, 3), :]
            dst1 = out_hbm.at[c * 8 + bh, pl.ds(vt0 + 3, 1), pl.ds(bl, 128)]
            src1 = obs.at[pl.ds(c * 4 + 3, 1), :]
            if do_issue:
                pltpu.async_copy(src3, dst3, sem)
                @pl.when(has4)
                def _():
                    pltpu.async_copy(src1, dst1, sem)
            else:
                pltpu.make_async_copy(src3, dst3, sem).wait()
                @pl.when(has4)
                def _():
                    pltpu.make_async_copy(src1, dst1, sem).wait()

    def step(b, slot, first):
        pltpu.make_async_copy(uv_hbm.at[idxf], g[slot], sg[slot]).wait()
        @pl.when(jnp.logical_not(first))
        def _():
            out_writes(b, slot, False)
        ga = g[slot]
        obs = ob[slot]
        for row in range(C * 4):
            orow = obs.at[row]
            for k in range(128 // L):
                s = pl.ds(k * L, L)
                p0 = pl.ds(row * 128 + k * L, L)
                p1 = pl.ds(FL + row * 128 + k * L, L)
                orow[s] = (ga[p0] + ga[p1]) * 0.5
        out_writes(b, slot, True)
        @pl.when(b + NSLOT < B)
        def _():
            issue(b + NSLOT, slot)

    for j in range(NSLOT):
        issue(j, j)

    def body(i, carry):
        b0 = i * NSLOT
        for j in range(NSLOT):
            step(b0 + j, j, i == 0)
        return carry

    lax.fori_loop(0, B // NSLOT, body, None)
    for j in range(NSLOT):
        out_writes(B - NSLOT + j, j, False)


def kernel(uv, uv_pixels):
    uv_flat = (uv.transpose(0, 3, 1, 2)
                 .reshape(B, C, H // 8, 4, 2, 2, 128)
                 .transpose(0, 1, 2, 4, 5, 3, 6)
                 .reshape(B * C * H * W))
    hp = uv_pixels.astype(jnp.int32)
    hw = jnp.stack([hp[:, 0, 0], hp[:, 0, 1], hp[:, 1, 0], hp[:, 1, 1]])
    hw = jnp.pad(hw, ((0, 0), (0, VPAD - V))).reshape(4 * VPAD)
    out = _uv2mesh_sc(uv_flat, hw)
    mesh = (out.reshape(C, 8, VT, 8, 128)
               .transpose(1, 3, 2, 4, 0)
               .reshape(B, VPAD, C))
    return mesh[:, :V, :]


# speedup vs baseline: 24.9579x; 1.0048x over previous
import functools

import jax
import jax.numpy as jnp
from jax import lax
from jax.experimental import pallas as pl
from jax.experimental.pallas import tpu as pltpu
from jax.experimental.pallas import tpu_sc as plsc

B = 64
H = 512
W = 256
C = 3
V = 14475
P = H * W
BSTRIDE = P * C

NC = 2
NS = 16
L = 16
NW = NC * NS

VT = 114
VPAD = VT * 128
NV = 512
FL = NV * C
NCH = FL // L
NSLOT = 8


@functools.partial(
    pl.kernel,
    out_type=jax.ShapeDtypeStruct((C * 8, VT, 8 * 128), jnp.float32),
    mesh=plsc.VectorSubcoreMesh(core_axis_name="c", subcore_axis_name="s",
                                num_cores=NC, num_subcores=NS),
    compiler_params=pltpu.CompilerParams(use_tc_tiling_on_sc=False,
                                         needs_layout_passes=False),
    scratch_types=(
        [pltpu.VMEM((4 * NV,), jnp.int32)]
        + [pltpu.VMEM((NV,), jnp.int32)] * 2
        + [pltpu.VMEM((2 * FL,), jnp.int32)]
        + [pltpu.VMEM((2 * FL,), jnp.float32)] * NSLOT
        + [pltpu.VMEM((C * 4, 128), jnp.float32)] * NSLOT
        + [pltpu.SemaphoreType.DMA] * NSLOT
        + [pltpu.SemaphoreType.DMA] * NSLOT
    ),
)
def _uv2mesh_sc(uv_hbm, hw_hbm, out_hbm, hw_v, idx0, idx1, idxf, *rest):
    g = rest[0:NSLOT]
    ob = rest[NSLOT:2 * NSLOT]
    sg = rest[2 * NSLOT:3 * NSLOT]
    so = rest[3 * NSLOT:4 * NSLOT]

    wid = lax.axis_index("s") * NC + lax.axis_index("c")
    vt0 = lax.shift_right_logical(wid * VT, 5)
    vt1 = lax.shift_right_logical((wid + 1) * VT, 5)
    has4 = (vt1 - vt0) == 4
    v0 = pl.multiple_of(lax.shift_left(vt0, 7), 128)

    for j in range(4):
        pltpu.sync_copy(hw_hbm.at[pl.ds(j * VPAD + v0, NV)],
                        hw_v.at[pl.ds(j * NV, NV)])

    for i in range(NV // L):
        s = pl.ds(i * L, L)
        h0 = hw_v[pl.ds(0 * NV + i * L, L)]
        w0 = hw_v[pl.ds(1 * NV + i * L, L)]
        h1 = hw_v[pl.ds(2 * NV + i * L, L)]
        w1 = hw_v[pl.ds(3 * NV + i * L, L)]
        t0 = (lax.shift_left(lax.shift_right_logical(h0, 3), 11)
              + lax.shift_left(h0 & 1, 10)
              + lax.shift_left(lax.shift_right_logical(w0, 7), 9)
              + lax.shift_left(lax.shift_right_logical(h0, 1) & 3, 7)
              + (w0 & 127))
        t1 = (lax.shift_left(lax.shift_right_logical(h1, 3), 11)
              + lax.shift_left(h1 & 1, 10)
              + lax.shift_left(lax.shift_right_logical(w1, 7), 9)
              + lax.shift_left(lax.shift_right_logical(h1, 1) & 3, 7)
              + (w1 & 127))
        idx0[s] = t0
        idx1[s] = t1

    for c in range(C):
        for i in range(NV // L):
            s = pl.ds(c * NV + i * L, L)
            s1 = pl.ds(FL + c * NV + i * L, L)
            vs = pl.ds(i * L, L)
            idxf[s] = idx0[vs] + (c * P)
            idxf[s1] = idx1[vs] + (c * P)

    def issue(b, slot):
        src = uv_hbm.at[pl.ds(b * BSTRIDE, BSTRIDE)]
        pltpu.async_copy(src.at[idxf], g[slot], sg[slot])

    def out_writes(b, slot, do_issue):
        bh = lax.shift_right_logical(b, 3)
        bl = (b & 7) * 128
        obs = ob[slot]
        sem = so[slot]
        for c in range(C):
            dst3 = out_hbm.at[c * 8 + bh, pl.ds(vt0, 3), pl.ds(bl, 128)]
            src3 = obs.at[pl.ds(c * 4, 3), :]
            dst1 = out_hbm.at[c * 8 + bh, pl.ds(vt0 + 3, 1), pl.ds(bl, 128)]
            src1 = obs.at[pl.ds(c * 4 + 3, 1), :]
            if do_issue:
                pltpu.async_copy(src3, dst3, sem)
                @pl.when(has4)
                def _():
                    pltpu.async_copy(src1, dst1, sem)
            else:
                pltpu.make_async_copy(src3, dst3, sem).wait()
                @pl.when(has4)
                def _():
                    pltpu.make_async_copy(src1, dst1, sem).wait()

    def step(b, slot, first):
        pltpu.make_async_copy(uv_hbm.at[idxf], g[slot], sg[slot]).wait()
        @pl.when(jnp.logical_not(first))
        def _():
            out_writes(b, slot, False)
        ga = g[slot]
        obs = ob[slot]
        for row in range(C * 4):
            orow = obs.at[row]
            for k in range(128 // L):
                s = pl.ds(k * L, L)
                p0 = pl.ds(row * 128 + k * L, L)
                p1 = pl.ds(FL + row * 128 + k * L, L)
                orow[s] = (ga[p0] + ga[p1]) * 0.5
        out_writes(b, slot, True)
        @pl.when(b + NSLOT < B)
        def _():
            issue(b + NSLOT, slot)

    for j in range(NSLOT):
        issue(j, j)

    def body(i, carry):
        b0 = i * NSLOT
        for j in range(NSLOT):
            step(b0 + j, j, i == 0)
        return carry

    lax.fori_loop(0, B // NSLOT, body, None)
    for j in range(NSLOT):
        out_writes(B - NSLOT + j, j, False)


def kernel(uv, uv_pixels):
    uv_flat = (uv.transpose(0, 3, 1, 2)
                 .reshape(B, C, H // 8, 4, 2, 2, 128)
                 .transpose(0, 1, 2, 4, 5, 3, 6)
                 .reshape(B * C * H * W))
    hp = uv_pixels.astype(jnp.int32)
    hw = jnp.stack([hp[:, 0, 0], hp[:, 0, 1], hp[:, 1, 0], hp[:, 1, 1]])
    hw = jnp.pad(hw, ((0, 0), (0, VPAD - V))).reshape(4 * VPAD)
    out = _uv2mesh_sc(uv_flat, hw)
    mesh = (out.reshape(C, 8, VT, 8, 128)
               .transpose(1, 3, 2, 4, 0)
               .reshape(B, VPAD, C))
    return mesh[:, :V, :]


# speedup vs baseline: 26.2343x; 1.0511x over previous
import functools

import jax
import jax.numpy as jnp
from jax import lax
from jax.experimental import pallas as pl
from jax.experimental.pallas import tpu as pltpu
from jax.experimental.pallas import tpu_sc as plsc

B = 64
H = 512
W = 256
C = 3
V = 14475
P = H * W
BSTRIDE = P * C

NC = 2
NS = 16
L = 16
NW = NC * NS

VT = 114
VPAD = VT * 128
NV = 512
FL = NV * C
NCH = FL // L
NSLOT = 8


@functools.partial(
    pl.kernel,
    out_type=jax.ShapeDtypeStruct((C * 8, VT, 8 * 128), jnp.float32),
    mesh=plsc.VectorSubcoreMesh(core_axis_name="c", subcore_axis_name="s",
                                num_cores=NC, num_subcores=NS),
    compiler_params=pltpu.CompilerParams(use_tc_tiling_on_sc=False,
                                         needs_layout_passes=False),
    scratch_types=(
        [pltpu.VMEM((4 * NV,), jnp.int32)]
        + [pltpu.VMEM((NV,), jnp.int32)] * 2
        + [pltpu.VMEM((2 * 3 * 384,), jnp.int32)]
        + [pltpu.VMEM((2 * 3 * 128,), jnp.int32)]
        + [pltpu.VMEM((2 * 3 * 384,), jnp.float32)] * NSLOT
        + [pltpu.VMEM((2 * 3 * 128,), jnp.float32)] * NSLOT
        + [pltpu.VMEM((C * 4, 128), jnp.float32)] * NSLOT
        + [pltpu.SemaphoreType.DMA] * NSLOT
        + [pltpu.SemaphoreType.DMA] * NSLOT
    ),
)
def _uv2mesh_sc(uv_hbm, hw_hbm, out_hbm, hw_v, idx0, idx1, idxfm, idxft,
                *rest):
    gm = rest[0:NSLOT]
    gt = rest[NSLOT:2 * NSLOT]
    ob = rest[2 * NSLOT:3 * NSLOT]
    sg = rest[3 * NSLOT:4 * NSLOT]
    so = rest[4 * NSLOT:5 * NSLOT]

    wid = lax.axis_index("s") * NC + lax.axis_index("c")
    vt0 = lax.shift_right_logical(wid * VT, 5)
    vt1 = lax.shift_right_logical((wid + 1) * VT, 5)
    has4 = (vt1 - vt0) == 4
    v0 = pl.multiple_of(lax.shift_left(vt0, 7), 128)

    for j in range(4):
        pltpu.sync_copy(hw_hbm.at[pl.ds(j * VPAD + v0, NV)],
                        hw_v.at[pl.ds(j * NV, NV)])

    for i in range(NV // L):
        s = pl.ds(i * L, L)
        h0 = hw_v[pl.ds(0 * NV + i * L, L)]
        w0 = hw_v[pl.ds(1 * NV + i * L, L)]
        h1 = hw_v[pl.ds(2 * NV + i * L, L)]
        w1 = hw_v[pl.ds(3 * NV + i * L, L)]
        t0 = (lax.shift_left(lax.shift_right_logical(h0, 3), 11)
              + lax.shift_left(h0 & 1, 10)
              + lax.shift_left(lax.shift_right_logical(w0, 7), 9)
              + lax.shift_left(lax.shift_right_logical(h0, 1) & 3, 7)
              + (w0 & 127))
        t1 = (lax.shift_left(lax.shift_right_logical(h1, 3), 11)
              + lax.shift_left(h1 & 1, 10)
              + lax.shift_left(lax.shift_right_logical(w1, 7), 9)
              + lax.shift_left(lax.shift_right_logical(h1, 1) & 3, 7)
              + (w1 & 127))
        idx0[s] = t0
        idx1[s] = t1

    for c in range(C):
        for i in range(384 // L):
            vs = pl.ds(i * L, L)
            idxfm[pl.ds(0 * 1152 + c * 384 + i * L, L)] = idx0[vs] + (c * P)
            idxfm[pl.ds(1 * 1152 + c * 384 + i * L, L)] = idx1[vs] + (c * P)
        for i in range(128 // L):
            vs = pl.ds(384 + i * L, L)
            idxft[pl.ds(0 * 384 + c * 128 + i * L, L)] = idx0[vs] + (c * P)
            idxft[pl.ds(1 * 384 + c * 128 + i * L, L)] = idx1[vs] + (c * P)

    def issue(b, slot):
        src = uv_hbm.at[pl.ds(b * BSTRIDE, BSTRIDE)]
        pltpu.async_copy(src.at[idxfm], gm[slot], sg[slot])
        @pl.when(has4)
        def _():
            pltpu.async_copy(src.at[idxft], gt[slot], sg[slot])

    def out_writes(b, slot, do_issue):
        bh = lax.shift_right_logical(b, 3)
        bl = (b & 7) * 128
        obs = ob[slot]
        sem = so[slot]
        for c in range(C):
            dst3 = out_hbm.at[c * 8 + bh, pl.ds(vt0, 3), pl.ds(bl, 128)]
            src3 = obs.at[pl.ds(c * 4, 3), :]
            dst1 = out_hbm.at[c * 8 + bh, pl.ds(vt0 + 3, 1), pl.ds(bl, 128)]
            src1 = obs.at[pl.ds(c * 4 + 3, 1), :]
            if do_issue:
                pltpu.async_copy(src3, dst3, sem)
                @pl.when(has4)
                def _():
                    pltpu.async_copy(src1, dst1, sem)
            else:
                pltpu.make_async_copy(src3, dst3, sem).wait()
                @pl.when(has4)
                def _():
                    pltpu.make_async_copy(src1, dst1, sem).wait()

    def step(b, slot, first):
        pltpu.make_async_copy(uv_hbm.at[idxfm], gm[slot], sg[slot]).wait()
        @pl.when(has4)
        def _():
            pltpu.make_async_copy(uv_hbm.at[idxft], gt[slot], sg[slot]).wait()
        @pl.when(jnp.logical_not(first))
        def _():
            out_writes(b, slot, False)
        ga = gm[slot]
        gb = gt[slot]
        obs = ob[slot]
        for c in range(C):
            for vt in range(4):
                orow = obs.at[c * 4 + vt]
                for k in range(128 // L):
                    s = pl.ds(k * L, L)
                    if vt < 3:
                        p0 = pl.ds(0 * 1152 + c * 384 + vt * 128 + k * L, L)
                        p1 = pl.ds(1 * 1152 + c * 384 + vt * 128 + k * L, L)
                        orow[s] = (ga[p0] + ga[p1]) * 0.5
                    else:
                        p0 = pl.ds(0 * 384 + c * 128 + k * L, L)
                        p1 = pl.ds(1 * 384 + c * 128 + k * L, L)
                        orow[s] = (gb[p0] + gb[p1]) * 0.5
        out_writes(b, slot, True)
        @pl.when(b + NSLOT < B)
        def _():
            issue(b + NSLOT, slot)

    for j in range(NSLOT):
        issue(j, j)

    def body(i, carry):
        b0 = i * NSLOT
        for j in range(NSLOT):
            step(b0 + j, j, i == 0)
        return carry

    lax.fori_loop(0, B // NSLOT, body, None)
    for j in range(NSLOT):
        out_writes(B - NSLOT + j, j, False)


def kernel(uv, uv_pixels):
    uv_flat = (uv.transpose(0, 3, 1, 2)
                 .reshape(B, C, H // 8, 4, 2, 2, 128)
                 .transpose(0, 1, 2, 4, 5, 3, 6)
                 .reshape(B * C * H * W))
    hp = uv_pixels.astype(jnp.int32)
    hw = jnp.stack([hp[:, 0, 0], hp[:, 0, 1], hp[:, 1, 0], hp[:, 1, 1]])
    hw = jnp.pad(hw, ((0, 0), (0, VPAD - V))).reshape(4 * VPAD)
    out = _uv2mesh_sc(uv_flat, hw)
    mesh = (out.reshape(C, 8, VT, 8, 128)
               .transpose(1, 3, 2, 4, 0)
               .reshape(B, VPAD, C))
    return mesh[:, :V, :]
